# baseline (speedup 1.0000x reference)
import functools

import jax
import jax.numpy as jnp
from jax import lax
from jax.experimental import pallas as pl
from jax.experimental.pallas import tpu as pltpu
from jax.experimental.pallas import tpu_sc as plsc

D = 64
B = 4096 * 50
NC, NS = 2, 16
NW = NC * NS
BPW = B // NW
CHUNK = 128
NCHUNK = BPW // CHUNK

_mesh = plsc.VectorSubcoreMesh(core_axis_name="c", subcore_axis_name="s")


@functools.partial(
    pl.kernel,
    mesh=_mesh,
    out_type=jax.ShapeDtypeStruct((B, D), jnp.float32),
    scratch_types=[
        pltpu.VMEM((NCHUNK, CHUNK), jnp.int32),
        pltpu.VMEM((CHUNK, D), jnp.float32),
        pltpu.SemaphoreType.DMA,
    ],
    compiler_params=pltpu.CompilerParams(use_tc_tiling_on_sc=False),
)
def _gather_kernel(idx_hbm, table_hbm, out_hbm, idx_v, rows_v, gsem):
    wid = lax.axis_index("s") * NC + lax.axis_index("c")
    base = wid * BPW
    pltpu.sync_copy(idx_hbm.at[wid], idx_v)

    def body(j, carry):
        pltpu.async_copy(table_hbm.at[idx_v.at[j]], rows_v, gsem).wait()
        pltpu.sync_copy(rows_v, out_hbm.at[pl.ds(base + j * CHUNK, CHUNK)])
        return carry

    lax.fori_loop(0, NCHUNK, body, 0)


def kernel(words, table):
    idx = words.reshape(NW, NCHUNK, CHUNK)
    out = _gather_kernel(idx, table)
    return out.reshape(4096, 50, D)


# speedup vs baseline: 1.0387x; 1.0387x over previous
import functools

import jax
import jax.numpy as jnp
from jax import lax
from jax.experimental import pallas as pl
from jax.experimental.pallas import tpu as pltpu
from jax.experimental.pallas import tpu_sc as plsc

D = 64
B = 4096 * 50
NC, NS = 2, 16
NW = NC * NS
BPW = B // NW
CHUNK = 128
NCHUNK = BPW // CHUNK

S = 5
SUP = S * CHUNK
NSUPER = BPW // SUP

_mesh = plsc.VectorSubcoreMesh(core_axis_name="c", subcore_axis_name="s")


@functools.partial(
    pl.kernel,
    mesh=_mesh,
    out_type=jax.ShapeDtypeStruct((B, D), jnp.float32),
    scratch_types=[
        pltpu.VMEM((NCHUNK, CHUNK), jnp.int32),
        pltpu.VMEM((SUP, D), jnp.float32),
        pltpu.VMEM((SUP, D), jnp.float32),
        pltpu.SemaphoreType.DMA,
        pltpu.SemaphoreType.DMA,
        pltpu.SemaphoreType.DMA,
        pltpu.SemaphoreType.DMA,
    ],
    compiler_params=pltpu.CompilerParams(use_tc_tiling_on_sc=False),
)
def _gather_kernel(idx_hbm, table_hbm, out_hbm, idx_v, buf0, buf1,
                   g0, g1, o0, o1):
    wid = lax.axis_index("s") * NC + lax.axis_index("c")
    base = wid * BPW
    pltpu.sync_copy(idx_hbm.at[wid], idx_v)

    bufs = (buf0, buf1)
    gsems = (g0, g1)
    osems = (o0, o1)

    def fire(s, buf, gsem):
        return [
            pltpu.async_copy(
                table_hbm.at[idx_v.at[s * S + c]],
                buf.at[pl.ds(c * CHUNK, CHUNK)],
                gsem,
            )
            for c in range(S)
        ]

    def drain_out(buf, osem):
        pltpu.make_async_copy(buf, out_hbm.at[pl.ds(base, SUP)], osem).wait()

    def body(p, carry):
        @pl.when(p > 0)
        def _():
            drain_out(buf0, o0)
            drain_out(buf1, o1)

        handles = [fire(2 * p + b, bufs[b], gsems[b]) for b in range(2)]
        for b in range(2):
            for h in handles[b]:
                h.wait()
            pltpu.async_copy(
                bufs[b],
                out_hbm.at[pl.ds(base + (2 * p + b) * SUP, SUP)],
                osems[b],
            )
        return carry

    lax.fori_loop(0, NSUPER // 2, body, 0)
    drain_out(buf0, o0)
    drain_out(buf1, o1)


def kernel(words, table):
    idx = words.reshape(NW, NCHUNK, CHUNK)
    out = _gather_kernel(idx, table)
    return out.reshape(4096, 50, D)
